# baseline (speedup 1.0000x reference)
import jax
import jax.numpy as jnp
from jax.experimental import pallas as pl
from jax.experimental.pallas import tpu as pltpu


def _vq_body(x_ref, cb_ref, onehot_ref, codebooks_ref, xa_ref, csq_ref):
    b = pl.program_id(0)
    cb = cb_ref[...]
    S, d = cb.shape
    L = x_ref.shape[1]
    A = xa_ref.shape[1]

    @pl.when(b == 0)
    def _init_csq():
        csq_ref[...] = jnp.broadcast_to(
            jnp.sum(cb * cb, axis=1)[None, :], csq_ref.shape)

    x = x_ref[0]
    xa_ref[:, :d] = x
    col8L = jax.lax.broadcasted_iota(jnp.int32, (L, A - d), 1)
    xa_ref[:, d:] = jnp.where(col8L == 0, 1.0, 0.0)

    cross = jax.lax.dot_general(
        x, cb, (((1,), (1,)), ((), ())),
        preferred_element_type=jnp.float32)
    x_sq = jnp.sum(x * x, axis=1, keepdims=True)
    d2 = x_sq - 2.0 * cross + csq_ref[0:1, :]
    deltas = jnp.argmin(d2, axis=1).astype(jnp.int32)
    col = jax.lax.broadcasted_iota(jnp.int32, (L, S), 1)
    onehot = (col == deltas[:, None]).astype(jnp.float32)
    onehot_ref[0] = onehot

    saug = jax.lax.dot_general(
        onehot, xa_ref[...], (((0,), (0,)), ((), ())),
        preferred_element_type=jnp.float32)
    counts = saug[:, d:d + 1]
    sums = saug[:, :d]
    codebooks_ref[0] = jnp.where(
        counts > 0.0, sums / jnp.maximum(counts, 1.0), cb)


def kernel(x, codebook):
    B, L, d = x.shape
    S = codebook.shape[0]
    A = d + 8
    onehot, codebooks = pl.pallas_call(
        _vq_body,
        grid=(B,),
        in_specs=[
            pl.BlockSpec((1, L, d), lambda b: (b, 0, 0)),
            pl.BlockSpec((S, d), lambda b: (0, 0)),
        ],
        out_specs=[
            pl.BlockSpec((1, L, S), lambda b: (b, 0, 0)),
            pl.BlockSpec((1, S, d), lambda b: (b, 0, 0)),
        ],
        out_shape=[
            jax.ShapeDtypeStruct((B, L, S), jnp.float32),
            jax.ShapeDtypeStruct((B, S, d), jnp.float32),
        ],
        scratch_shapes=[
            pltpu.VMEM((L, A), jnp.float32),
            pltpu.VMEM((8, S), jnp.float32),
        ],
    )(x, codebook)
    return onehot, codebooks
